# Initial kernel scaffold; baseline (speedup 1.0000x reference)
import jax
import jax.numpy as jnp
from jax.experimental import pallas as pl


def kernel(feature_teacher, scores, labels, lda_weight, lda_bias, cluster_centers, teacher_scores):
    raise NotImplementedError("write your pallas kernel here")


# speedup vs baseline: 7.4717x; 7.4717x over previous
import functools

import jax
import jax.numpy as jnp
from jax.experimental import pallas as pl
from jax.experimental.pallas import tpu as pltpu

B = 16
C_IN = 768
HW2 = 1024
K_SUB = 32
LDA_COMP = 128
S_OUT = 128
N_TOT = B * HW2


def _tc_kernel(labels_ref, x_ref, sp_ref, w_ref, bias_ref, cc_ref, ts_ref, out_ref):
    b = pl.program_id(0)
    label = labels_ref[b]

    x = x_ref[0]
    sp = sp_ref[0]
    w = w_ref[...]
    bias = bias_ref[...]
    cg = cc_ref[pl.ds(label * K_SUB, K_SUB), :]
    tb = ts_ref[pl.ds(label * K_SUB, K_SUB), :]

    m = jnp.dot(cg, w, preferred_element_type=jnp.float32)
    a = jnp.dot(m, x, preferred_element_type=jnp.float32)
    q = (jnp.sum(cg * cg, axis=1, keepdims=True)
         - 2.0 * jnp.dot(cg, bias.T, preferred_element_type=jnp.float32))
    score = q - 2.0 * a

    minv = jnp.min(score, axis=0, keepdims=True)
    kio = jax.lax.broadcasted_iota(jnp.int32, (K_SUB, HW2), 0)
    idx = jnp.min(jnp.where(score == minv, kio, K_SUB), axis=0, keepdims=True)
    onehot = (kio == idx).astype(jnp.float32)

    counts = jnp.sum(onehot, axis=1, keepdims=True)
    u = jax.lax.dot_general(onehot, sp, (((1,), (1,)), ((), ())),
                            preferred_element_type=jnp.float32)

    m0 = jnp.max(sp, axis=0, keepdims=True)
    lse = m0 + jnp.log(jnp.sum(jnp.exp(sp - m0), axis=0, keepdims=True))
    sum_lse = jnp.sum(lse, keepdims=True).reshape(1, 1)

    e_blk = jnp.sum(tb * jnp.log(tb), axis=1, keepdims=True)
    loss_b = (jnp.sum(counts * e_blk, keepdims=True).reshape(1, 1)
              - jnp.sum(u * tb, keepdims=True).reshape(1, 1) + sum_lse)

    @pl.when(b == 0)
    def _():
        out_ref[...] = jnp.zeros_like(out_ref)

    out_ref[...] += loss_b


@jax.jit
def kernel(feature_teacher, scores, labels, lda_weight, lda_bias,
           cluster_centers, teacher_scores):
    x = feature_teacher.reshape(B, C_IN, HW2)
    sp = scores.reshape(B, S_OUT, HW2)
    bias2 = lda_bias.reshape(1, LDA_COMP)
    labels32 = labels.astype(jnp.int32)

    grid_spec = pltpu.PrefetchScalarGridSpec(
        num_scalar_prefetch=1,
        grid=(B,),
        in_specs=[
            pl.BlockSpec((1, C_IN, HW2), lambda b, L: (b, 0, 0)),
            pl.BlockSpec((1, S_OUT, HW2), lambda b, L: (b, 0, 0)),
            pl.BlockSpec((LDA_COMP, C_IN), lambda b, L: (0, 0)),
            pl.BlockSpec((1, LDA_COMP), lambda b, L: (0, 0)),
            pl.BlockSpec((B * 256, LDA_COMP), lambda b, L: (0, 0)),
            pl.BlockSpec((B * 256, S_OUT), lambda b, L: (0, 0)),
        ],
        out_specs=pl.BlockSpec((1, 1), lambda b, L: (0, 0)),
    )
    total = pl.pallas_call(
        _tc_kernel,
        grid_spec=grid_spec,
        out_shape=jax.ShapeDtypeStruct((1, 1), jnp.float32),
    )(labels32, x, sp, lda_weight, bias2, cluster_centers, teacher_scores)
    return total[0, 0] / N_TOT
